# baseline (speedup 1.0000x reference)
import functools

import jax
import jax.numpy as jnp
from jax import lax
from jax.experimental import pallas as pl
from jax.experimental.pallas import tpu as pltpu
from jax.experimental.pallas import tpu_sc as plsc

_NC = 2
_NS = 16
_NW = _NC * _NS

_LB = 128


def _emb_gather_t(x_t, table_pk):
  hist, batch = x_t.shape
  n_pk = table_pk.shape[0]
  emb_dim = 32
  blocks_per_w = batch // _LB // _NW
  mesh = plsc.VectorSubcoreMesh(core_axis_name="c", subcore_axis_name="s")

  @functools.partial(
      pl.kernel,
      out_type=jax.ShapeDtypeStruct((hist, emb_dim, batch), jnp.float32),
      mesh=mesh,
      scratch_types=[
          pltpu.VMEM((_LB,), jnp.int32),
          pltpu.VMEM((_LB,), jnp.int32),
          pltpu.VMEM((_LB,), jnp.int32),
          pltpu.VMEM((_LB, 128), jnp.float32),
          pltpu.VMEM((emb_dim, _LB), jnp.float32),
          pltpu.SemaphoreType.DMA,
      ],
      compiler_params=pltpu.CompilerParams(needs_layout_passes=False),
  )
  def k(xt_hbm, tab_hbm, out_hbm, idx_v, rowid_v, base_v, fetch_v, col_v,
        sem):
    wid = lax.axis_index("s") * _NC + lax.axis_index("c")

    def unit(h, jb):
      b0 = (wid * blocks_per_w + jb) * _LB
      pltpu.sync_copy(xt_hbm.at[h, pl.ds(b0, _LB)], idx_v)
      for t in range(_LB // 16):
        v = idx_v[pl.ds(t * 16, 16)]
        rowid_v[pl.ds(t * 16, 16)] = lax.shift_right_logical(v, 2)
        base_v[pl.ds(t * 16, 16)] = (v & 3) * emb_dim
      pltpu.async_copy(tab_hbm.at[rowid_v], fetch_v, sem).wait()
      for j16 in range(_LB // 16):
        rows16 = j16 * 16 + lax.iota(jnp.int32, 16)
        bases = base_v[pl.ds(j16 * 16, 16)]
        for f in range(emb_dim):
          vals = plsc.load_gather(fetch_v, [rows16, bases + f])
          col_v[f, pl.ds(j16 * 16, 16)] = vals
      pltpu.sync_copy(col_v, out_hbm.at[h, :, pl.ds(b0, _LB)])

    def body(u, carry):
      unit(u % hist, u // hist)
      return carry

    lax.fori_loop(0, hist * blocks_per_w, body, 0)

  return k(x_t, table_pk)


def kernel(x, table):
  nrow, dim = table.shape
  x_t = jnp.transpose(x).astype(jnp.int32)
  table_pk = table.reshape(nrow * dim // 128, 128)
  out_t = _emb_gather_t(x_t, table_pk)
  return jnp.transpose(out_t, (2, 0, 1))


# speedup vs baseline: 1.2376x; 1.2376x over previous
import functools

import jax
import jax.numpy as jnp
from jax import lax
from jax.experimental import pallas as pl
from jax.experimental.pallas import tpu as pltpu
from jax.experimental.pallas import tpu_sc as plsc

_NC = 2
_NS = 16
_NW = _NC * _NS

_LB = 128


def _emb_gather_t(x_t, table_pk):
  hist, batch = x_t.shape
  n_pk = table_pk.shape[0]
  emb_dim = 32
  blocks_per_w = batch // _LB // _NW
  mesh = plsc.VectorSubcoreMesh(core_axis_name="c", subcore_axis_name="s")

  @functools.partial(
      pl.kernel,
      out_type=jax.ShapeDtypeStruct((hist, emb_dim, batch), jnp.float32),
      mesh=mesh,
      scratch_types=(
          [pltpu.VMEM((_LB,), jnp.int32) for _ in range(2)]
          + [pltpu.VMEM((_LB,), jnp.int32) for _ in range(2)]
          + [pltpu.VMEM((_LB,), jnp.int32) for _ in range(2)]
          + [pltpu.VMEM((_LB, 128), jnp.float32) for _ in range(2)]
          + [pltpu.VMEM((emb_dim, _LB), jnp.float32) for _ in range(2)]
          + [pltpu.SemaphoreType.DMA for _ in range(4)]
      ),
      compiler_params=pltpu.CompilerParams(needs_layout_passes=False),
  )
  def k(xt_hbm, tab_hbm, out_hbm, *refs):
    idx_v = refs[0:2]
    rowid_v = refs[2:4]
    base_v = refs[4:6]
    fetch_v = refs[6:8]
    col_v = refs[8:10]
    gsem = refs[10:12]
    wsem = refs[12:14]
    wid = lax.axis_index("s") * _NC + lax.axis_index("c")
    n_units = hist * blocks_per_w

    def unit_coords(u):
      h = u % hist
      b0 = (wid * blocks_per_w + u // hist) * _LB
      return h, b0

    def stage_and_fire(u, s):
      h, b0 = unit_coords(u)
      pltpu.sync_copy(xt_hbm.at[h, pl.ds(b0, _LB)], idx_v[s])
      for t in range(_LB // 16):
        v = idx_v[s][pl.ds(t * 16, 16)]
        rowid_v[s][pl.ds(t * 16, 16)] = lax.shift_right_logical(v, 2)
        base_v[s][pl.ds(t * 16, 16)] = (v & 3) * emb_dim
      pltpu.async_copy(tab_hbm.at[rowid_v[s]], fetch_v[s], gsem[s])

    def wait_gather(s):
      pltpu.make_async_copy(tab_hbm.at[rowid_v[s]], fetch_v[s], gsem[s]).wait()

    def wait_write(s):
      pltpu.make_async_copy(
          col_v[s], out_hbm.at[0, :, pl.ds(0, _LB)], wsem[s]).wait()

    def compact_and_write(u, s):
      for j16 in range(_LB // 16):
        rows16 = j16 * 16 + lax.iota(jnp.int32, 16)
        bases = base_v[s][pl.ds(j16 * 16, 16)]
        for f in range(emb_dim):
          vals = plsc.load_gather(fetch_v[s], [rows16, bases + f])
          col_v[s][f, pl.ds(j16 * 16, 16)] = vals
      h, b0 = unit_coords(u)
      pltpu.async_copy(col_v[s], out_hbm.at[h, :, pl.ds(b0, _LB)], wsem[s])

    stage_and_fire(0, 0)

    def body(i, carry):
      for s in (0, 1):
        u = 2 * i + s
        nxt = 1 - s

        @pl.when(u + 1 < n_units)
        def _():
          stage_and_fire(u + 1, nxt)

        wait_gather(s)

        @pl.when(u >= 2)
        def _():
          wait_write(s)

        compact_and_write(u, s)
      return carry

    lax.fori_loop(0, n_units // 2, body, 0)
    wait_write(0)
    wait_write(1)

  return k(x_t, table_pk)


def kernel(x, table):
  nrow, dim = table.shape
  x_t = jnp.transpose(x).astype(jnp.int32)
  table_pk = table.reshape(nrow * dim // 128, 128)
  out_t = _emb_gather_t(x_t, table_pk)
  return jnp.transpose(out_t, (2, 0, 1))
